# baseline (speedup 1.0000x reference)
import functools

import jax
import jax.numpy as jnp
from jax import lax
from jax.experimental import pallas as pl
from jax.experimental.pallas import tpu as pltpu
from jax.experimental.pallas import tpu_sc as plsc

_TEMPERATURE = 11.3137
_NUM_SAMPLED = 1024
_EPS = 1e-06

_B = 4
_S = 4096
_NEARLY = _NUM_SAMPLED + 1
_ETILE = 128
_NET = 9


def _rank_kernel(sig_ref, rank_ref):
    et = pl.program_id(1)
    row = sig_ref[0, 0, :]
    ev = sig_ref[0, 0, pl.ds(et * _ETILE, _ETILE)]
    eidx = et * _ETILE + lax.broadcasted_iota(jnp.int32, (_ETILE, 1), 0)
    jidx = lax.broadcasted_iota(jnp.int32, (_ETILE, _S), 1)
    allv = row[None, :]
    evc = ev[:, None]
    lt = allv < evc
    eq_before = (allv == evc) & (jidx < eidx)
    cnt = jnp.sum((lt | eq_before).astype(jnp.int32), axis=1)
    rank_ref[0, 0, :] = cnt


@functools.lru_cache(maxsize=1)
def _make_scatter_kernel():
    mesh = plsc.VectorSubcoreMesh(core_axis_name="c", subcore_axis_name="s")

    @functools.partial(
        pl.kernel,
        mesh=mesh,
        out_type=jax.ShapeDtypeStruct((_B, _S), jnp.float32),
        scratch_types=[
            pltpu.VMEM((_NET * _ETILE,), jnp.int32),
            pltpu.VMEM((_S,), jnp.float32),
        ],
        compiler_params=pltpu.CompilerParams(needs_layout_passes=False),
    )
    def scatter_kernel(ranks_hbm, out_hbm, ranks_v, row_v):
        wid = lax.axis_index("s") * 2 + lax.axis_index("c")

        @pl.when(wid < _B)
        def _():
            pltpu.sync_copy(ranks_hbm.at[wid], ranks_v)
            zeros16 = jnp.zeros((16,), jnp.float32)

            def zero_body(i, _):
                row_v[pl.ds(i * 16, 16)] = zeros16
                return 0

            lax.fori_loop(0, _S // 16, zero_body, 0)

            ones16 = jnp.ones((16,), jnp.float32)
            lane = lax.iota(jnp.int32, 16)

            def scat_body(i, _):
                r16 = ranks_v[pl.ds(i * 16, 16)]
                pos = i * 16 + lane
                valid = pos < _NEARLY
                plsc.store_scatter(row_v, [r16 + 1], ones16, mask=valid)
                return 0

            lax.fori_loop(0, _NET * _ETILE // 16, scat_body, 0)

            head = row_v[pl.ds(0, 16)]
            row_v[pl.ds(0, 16)] = jnp.where(lane == 0, 1.0, head)
            pltpu.sync_copy(row_v, out_hbm.at[wid])

    return scatter_kernel


def kernel(q, k, v, token_mask):
    attn = jnp.matmul(q[..., :1, :], jnp.swapaxes(k, -2, -1)) / _TEMPERATURE
    attn = attn - jnp.max(attn, axis=-1, keepdims=True)
    batch_size, seq_length = token_mask.shape
    attn_mask = token_mask.reshape(batch_size, 1, 1, seq_length)
    attn = jnp.exp(attn) * attn_mask
    attn = (attn + _EPS / seq_length) / (jnp.sum(attn, axis=-1, keepdims=True) + _EPS)
    sig = jnp.sum(attn[:, :, 0], axis=1)
    sig = sig[:, 1:]
    sig = sig / jnp.sum(sig, axis=1, keepdims=True)

    sigp = jnp.concatenate(
        [sig, jnp.full((batch_size, 1), jnp.inf, jnp.float32)], axis=1
    ).reshape(_B, 1, _S)

    ranks = pl.pallas_call(
        _rank_kernel,
        grid=(_B, _NET),
        in_specs=[pl.BlockSpec((1, 1, _S), lambda b, e: (b, 0, 0))],
        out_specs=pl.BlockSpec((1, 1, _ETILE), lambda b, e: (b, 0, e)),
        out_shape=jax.ShapeDtypeStruct((_B, 1, _NET * _ETILE), jnp.int32),
    )(sigp)

    onehot = _make_scatter_kernel()(ranks.reshape(_B, _NET * _ETILE))
    return onehot != 0.0
